# baseline (speedup 1.0000x reference)
import functools

import jax
import jax.numpy as jnp
from jax import lax
from jax.experimental import pallas as pl
from jax.experimental.pallas import tpu as pltpu
from jax.experimental.pallas import tpu_sc as plsc

_L = 16


def _aligner_body(nBatch, outDim, nWin, nSeq,
                  trf_hbm, src_hbm, ov_hbm, out_hbm,
                  trf_v, src_v, ov_v, acc_v):
    nLen = 2 * nSeq
    accN = nLen + nWin
    cid = lax.axis_index("c")
    sid = lax.axis_index("s")
    wid = sid * 2 + cid
    jobs_per_worker = (nBatch * outDim) // 32
    cblocks = 32 // nBatch
    b = wid // cblocks
    c0 = (wid % cblocks) * jobs_per_worker

    pltpu.sync_copy(src_hbm.at[b], src_v)
    pltpu.sync_copy(ov_hbm, ov_v)
    ovec = ov_v[...]

    def job(j, _):
        c = c0 + j
        pltpu.sync_copy(trf_hbm.at[b, c], trf_v)

        def init(i, _):
            acc_v[pl.ds(i * _L, _L)] = ovec
            return _
        lax.fori_loop(0, accN // _L, init, None)

        def sgroup(sb, _):
            tvec = src_v[pl.ds(sb * _L, _L)]
            vals = [trf_v[w, pl.ds(sb * _L, _L)] for w in range(nWin)]
            idxs = [tvec + r for r in range(8)]
            for w in range(nWin):
                base = (w // 8) * 8
                plsc.addupdate_scatter(
                    acc_v.at[pl.ds(base, accN - base)], [idxs[w % 8]], vals[w])
            return _
        lax.fori_loop(0, nSeq // _L, sgroup, None)

        pltpu.sync_copy(acc_v.at[pl.ds(0, nLen)], out_hbm.at[b, c])
        return _
    lax.fori_loop(0, jobs_per_worker, job, None)


def kernel(TRFs, sourceIdx, nRealLen):
    nBatch, outDim, nWin, nSeq = TRFs.shape
    nLen = 2 * nSeq
    accN = nLen + nWin

    maxSrc = jnp.max(sourceIdx[:, -1])
    overflow = jnp.maximum(maxSrc + 1 - nRealLen, 0).astype(jnp.float32)
    ov_arr = jnp.broadcast_to(overflow, (_L,))

    mesh = plsc.VectorSubcoreMesh(core_axis_name="c", subcore_axis_name="s")
    run = pl.kernel(
        functools.partial(_aligner_body, nBatch, outDim, nWin, nSeq),
        mesh=mesh,
        compiler_params=pltpu.CompilerParams(needs_layout_passes=False),
        out_type=jax.ShapeDtypeStruct((nBatch, outDim, nLen), jnp.float32),
        scratch_types=[
            pltpu.VMEM((nWin, nSeq), jnp.float32),
            pltpu.VMEM((nSeq,), jnp.int32),
            pltpu.VMEM((_L,), jnp.float32),
            pltpu.VMEM((accN,), jnp.float32),
        ],
    )
    return run(TRFs, sourceIdx, ov_arr)


# speedup vs baseline: 1.4108x; 1.4108x over previous
import functools

import jax
import jax.numpy as jnp
from jax import lax
from jax.experimental import pallas as pl
from jax.experimental.pallas import tpu as pltpu
from jax.experimental.pallas import tpu_sc as plsc

_L = 16


def _aligner_body(nBatch, outDim, nWin, nSeq,
                  trf_hbm, src_hbm, ov_hbm, out_hbm,
                  buf_a, buf_b, src_v, ov_v, acc_v, sem_a, sem_b):
    nLen = 2 * nSeq
    accN = nLen + nWin
    half = nSeq // 2
    cid = lax.axis_index("c")
    sid = lax.axis_index("s")
    wid = sid * 2 + cid
    jobs_per_worker = (nBatch * outDim) // 32
    cblocks = 32 // nBatch
    b = wid // cblocks
    c0 = (wid % cblocks) * jobs_per_worker

    pltpu.sync_copy(src_hbm.at[b], src_v)
    pltpu.sync_copy(ov_hbm, ov_v)
    ovec = ov_v[...]

    def in_copy(c, h, buf, sem):
        return pltpu.make_async_copy(
            trf_hbm.at[b, c, :, pl.ds(h * half, half)], buf, sem)

    def compute(buf, s_base):
        def sgroup(sb, _):
            tvec = src_v[pl.ds(s_base + sb * _L, _L)]
            vals = [buf[w, pl.ds(sb * _L, _L)] for w in range(nWin)]
            idxs = [tvec + r for r in range(8)]
            for w in range(nWin):
                base = (w // 8) * 8
                plsc.addupdate_scatter(
                    acc_v.at[pl.ds(base, accN - base)], [idxs[w % 8]], vals[w])
            return _
        lax.fori_loop(0, half // _L, sgroup, None)

    in_copy(c0, 0, buf_a, sem_a).start()

    def job(j, _):
        c = c0 + j

        def init(i, _):
            acc_v[pl.ds(i * _L, _L)] = ovec
            return _
        lax.fori_loop(0, accN // _L, init, None)

        in_copy(c, 0, buf_a, sem_a).wait()
        in_copy(c, 1, buf_b, sem_b).start()
        compute(buf_a, 0)
        in_copy(c, 1, buf_b, sem_b).wait()

        @pl.when(j < jobs_per_worker - 1)
        def _prefetch():
            in_copy(c + 1, 0, buf_a, sem_a).start()

        compute(buf_b, half)
        pltpu.sync_copy(acc_v.at[pl.ds(0, nLen)], out_hbm.at[b, c])
        return _
    lax.fori_loop(0, jobs_per_worker, job, None)


def kernel(TRFs, sourceIdx, nRealLen):
    nBatch, outDim, nWin, nSeq = TRFs.shape
    nLen = 2 * nSeq
    accN = nLen + nWin

    maxSrc = jnp.max(sourceIdx[:, -1])
    overflow = jnp.maximum(maxSrc + 1 - nRealLen, 0).astype(jnp.float32)
    ov_arr = jnp.broadcast_to(overflow, (_L,))

    mesh = plsc.VectorSubcoreMesh(core_axis_name="c", subcore_axis_name="s")
    run = pl.kernel(
        functools.partial(_aligner_body, nBatch, outDim, nWin, nSeq),
        mesh=mesh,
        compiler_params=pltpu.CompilerParams(needs_layout_passes=False),
        out_type=jax.ShapeDtypeStruct((nBatch, outDim, nLen), jnp.float32),
        scratch_types=[
            pltpu.VMEM((nWin, nSeq // 2), jnp.float32),
            pltpu.VMEM((nWin, nSeq // 2), jnp.float32),
            pltpu.VMEM((nSeq,), jnp.int32),
            pltpu.VMEM((_L,), jnp.float32),
            pltpu.VMEM((accN,), jnp.float32),
            pltpu.SemaphoreType.DMA,
            pltpu.SemaphoreType.DMA,
        ],
    )
    return run(TRFs, sourceIdx, ov_arr)
